# Initial kernel scaffold; baseline (speedup 1.0000x reference)
import jax
import jax.numpy as jnp
from jax.experimental import pallas as pl


def kernel(batch, x, edge_index, rel_embed, W_in, W_out, W_loop, gamma, beta):
    raise NotImplementedError("write your pallas kernel here")


# speedup vs baseline: 73.7563x; 73.7563x over previous
import functools

import jax
import jax.numpy as jnp
from jax import lax
from jax.experimental import pallas as pl
from jax.experimental.pallas import tpu as pltpu
from jax.experimental.pallas import tpu_sc as plsc

N = 10000
D = 128
E = 320000
EH = E // 2
NCORE = 2
NSUB = 16
EPT = EH // NSUB
CHUNK = 128
NCHUNK = 80
EPTP = NCHUNK * CHUNK
NP = 10240
NPT = NP // NSUB
PADIDX = NP - 1


def _rsqrt_newton(d):
    y = jnp.full((16,), 1.0 / 512.0, jnp.float32)
    for _ in range(22):
        y = y * (1.5 - 0.5 * d * y * y)
    return jnp.where(d > 0.5, y, 0.0)


@functools.partial(
    pl.kernel,
    mesh=plsc.VectorSubcoreMesh(core_axis_name="c", subcore_axis_name="s"),
    out_type=jax.ShapeDtypeStruct((NCORE, NP), jnp.float32),
    compiler_params=pltpu.CompilerParams(needs_layout_passes=False),
    scratch_types=[
        pltpu.VMEM((NCHUNK, CHUNK), jnp.int32),
        pltpu.VMEM((EPTP,), jnp.int32),
        pltpu.VMEM((EPTP,), jnp.float32),
        pltpu.VMEM((NP,), jnp.float32),
        pltpu.VMEM((NPT,), jnp.float32),
        pltpu.VMEM((NPT,), jnp.float32),
        pltpu.VMEM_SHARED((NP,), jnp.float32),
        pltpu.VMEM_SHARED((NP,), jnp.float32),
        pltpu.VMEM_SHARED((NP,), jnp.float32),
    ],
)
def _sc_coeffs(row_hbm, col_hbm, out_hbm, row_buf, col_buf, vals_buf,
               dinv_full, slice_buf, dinv_slice, sp_deg, sp_t, sp_dinv):
    cid = lax.axis_index("c")
    sid = lax.axis_index("s")
    nbase = sid * NPT

    pltpu.sync_copy(row_hbm.at[cid, sid], row_buf)
    pltpu.sync_copy(col_hbm.at[cid, sid], col_buf)

    zeros = jnp.zeros((16,), jnp.float32)
    ones = jnp.ones((16,), jnp.float32)

    def zbody(i, c):
        slice_buf[pl.ds(i * 16, 16)] = zeros
        return c
    lax.fori_loop(0, NPT // 16, zbody, 0)
    pltpu.sync_copy(slice_buf, sp_deg.at[pl.ds(nbase, NPT)])
    pltpu.sync_copy(slice_buf, sp_t.at[pl.ds(nbase, NPT)])

    def obody(i, c):
        vals_buf[pl.ds(i * 16, 16)] = ones
        return c
    lax.fori_loop(0, EPTP // 16, obody, 0)

    plsc.subcore_barrier()

    def hbody(k, c):
        pltpu.sync_copy(vals_buf.at[pl.ds(k * CHUNK, CHUNK)],
                        sp_deg.at[row_buf.at[k]], add=True)
        return c
    lax.fori_loop(0, NCHUNK, hbody, 0)

    plsc.subcore_barrier()

    pltpu.sync_copy(sp_deg.at[pl.ds(nbase, NPT)], slice_buf)

    def dbody(i, c):
        d = slice_buf[pl.ds(i * 16, 16)]
        dinv_slice[pl.ds(i * 16, 16)] = _rsqrt_newton(d)
        return c
    lax.fori_loop(0, NPT // 16, dbody, 0)
    pltpu.sync_copy(dinv_slice, sp_dinv.at[pl.ds(nbase, NPT)])

    plsc.subcore_barrier()

    pltpu.sync_copy(sp_dinv, dinv_full)

    def gbody(i, c):
        cidx = col_buf[pl.ds(i * 16, 16)]
        vals_buf[pl.ds(i * 16, 16)] = plsc.load_gather(dinv_full, [cidx])
        return c
    lax.fori_loop(0, EPTP // 16, gbody, 0)

    def sbody(k, c):
        pltpu.sync_copy(vals_buf.at[pl.ds(k * CHUNK, CHUNK)],
                        sp_t.at[row_buf.at[k]], add=True)
        return c
    lax.fori_loop(0, NCHUNK, sbody, 0)

    plsc.subcore_barrier()

    pltpu.sync_copy(sp_t.at[pl.ds(nbase, NPT)], slice_buf)

    def fbody(i, c):
        t = slice_buf[pl.ds(i * 16, 16)]
        dv = dinv_slice[pl.ds(i * 16, 16)]
        slice_buf[pl.ds(i * 16, 16)] = t * dv
        return c
    lax.fori_loop(0, NPT // 16, fbody, 0)
    pltpu.sync_copy(slice_buf, out_hbm.at[cid, pl.ds(nbase, NPT)])


def _tc_body(x_ref, win_ref, wout_ref, wloop_ref, sin_ref, sout_ref,
             g_ref, b_ref, o_ref):
    x = x_ref[...]
    pre = (jnp.dot(x, win_ref[...], preferred_element_type=jnp.float32) * sin_ref[...]
           + jnp.dot(x, wout_ref[...], preferred_element_type=jnp.float32) * sout_ref[...]
           + jnp.dot(x, wloop_ref[...], preferred_element_type=jnp.float32)
           ) * jnp.float32(1.0 / 3.0)
    mean = jnp.mean(pre, axis=0, keepdims=True)
    var = jnp.mean(pre * pre, axis=0, keepdims=True) - mean * mean
    inv = lax.rsqrt(var + 1e-5)
    o_ref[...] = jnp.tanh(g_ref[...] * (pre - mean) * inv + b_ref[...])


def kernel(batch, x, edge_index, rel_embed, W_in, W_out, W_loop, gamma, beta):
    rows = edge_index[0].reshape(NCORE, NSUB, EPT)
    cols = edge_index[1].reshape(NCORE, NSUB, EPT)
    pad = jnp.full((NCORE, NSUB, EPTP - EPT), PADIDX, jnp.int32)
    row_t = jnp.concatenate([rows, pad], axis=-1).reshape(NCORE, NSUB, NCHUNK, CHUNK)
    col_t = jnp.concatenate([cols, pad], axis=-1)

    s2 = _sc_coeffs(row_t, col_t)
    sin = s2[0, :N].reshape(N, 1)
    sout = s2[1, :N].reshape(N, 1)

    out = pl.pallas_call(
        _tc_body,
        out_shape=jax.ShapeDtypeStruct((N, D), jnp.float32),
    )(x, W_in, W_out, W_loop, sin, sout, gamma.reshape(1, D), beta.reshape(1, D))
    return out, rel_embed


# speedup vs baseline: 97.5626x; 1.3228x over previous
import functools

import jax
import jax.numpy as jnp
from jax import lax
from jax.experimental import pallas as pl
from jax.experimental.pallas import tpu as pltpu
from jax.experimental.pallas import tpu_sc as plsc

N = 10000
D = 128
E = 320000
EH = E // 2
NCORE = 2
NSUB = 16
EPT = EH // NSUB
CHUNK = 128
NCHUNK = 80
EPTP = NCHUNK * CHUNK
NP = 10240
NPT = NP // NSUB
PADIDX = NP - 1
GRP = 20
NGRP = NCHUNK // GRP


def _rsqrt_newton(d):
    seed = jnp.where(d < 64.0, 1.0 / 8.0,
                     jnp.where(d < 4096.0, 1.0 / 64.0, 1.0 / 512.0))
    y = seed.astype(jnp.float32)
    for _ in range(11):
        y = y * (1.5 - 0.5 * d * y * y)
    return jnp.where(d > 0.5, y, 0.0)


@functools.partial(
    pl.kernel,
    mesh=plsc.VectorSubcoreMesh(core_axis_name="c", subcore_axis_name="s"),
    out_type=jax.ShapeDtypeStruct((NCORE, NP), jnp.float32),
    compiler_params=pltpu.CompilerParams(needs_layout_passes=False),
    scratch_types=[
        pltpu.VMEM((NCHUNK, CHUNK), jnp.int32),
        pltpu.VMEM((EPTP,), jnp.int32),
        pltpu.VMEM((EPTP,), jnp.float32),
        pltpu.VMEM((NP,), jnp.float32),
        pltpu.VMEM((NPT,), jnp.float32),
        pltpu.VMEM((NPT,), jnp.float32),
        pltpu.VMEM_SHARED((NP,), jnp.float32),
        pltpu.VMEM_SHARED((NP,), jnp.float32),
        pltpu.VMEM_SHARED((NP,), jnp.float32),
        pltpu.SemaphoreType.DMA,
        pltpu.SemaphoreType.DMA,
    ],
)
def _sc_coeffs(row_hbm, col_hbm, out_hbm, row_buf, col_buf, vals_buf,
               dinv_full, slice_buf, dinv_slice, sp_deg, sp_t, sp_dinv,
               sem_in, sem_scat):
    cid = lax.axis_index("c")
    sid = lax.axis_index("s")
    nbase = sid * NPT

    pltpu.async_copy(row_hbm.at[cid, sid], row_buf, sem_in)
    pltpu.async_copy(col_hbm.at[cid, sid], col_buf, sem_in)

    zeros = jnp.zeros((16,), jnp.float32)
    ones = jnp.ones((16,), jnp.float32)

    def zbody(i, c):
        slice_buf[pl.ds(i * 16, 16)] = zeros
        return c
    lax.fori_loop(0, NPT // 16, zbody, 0)
    pltpu.sync_copy(slice_buf, sp_deg.at[pl.ds(nbase, NPT)])
    pltpu.sync_copy(slice_buf, sp_t.at[pl.ds(nbase, NPT)])

    def obody(i, c):
        vals_buf[pl.ds(i * 16, 16)] = ones
        return c
    lax.fori_loop(0, EPTP // 16, obody, 0)

    pltpu.make_async_copy(row_hbm.at[cid, sid], row_buf, sem_in).wait()
    pltpu.make_async_copy(col_hbm.at[cid, sid], col_buf, sem_in).wait()

    plsc.subcore_barrier()

    def _scatter_all(target):
        def grp_body(g, c):
            def start_body(j, c2):
                k = g * GRP + j
                pltpu.async_copy(vals_buf.at[pl.ds(k * CHUNK, CHUNK)],
                                 target.at[row_buf.at[k]], sem_scat, add=True)
                return c2
            lax.fori_loop(0, GRP, start_body, 0)

            def drain_body(j, c2):
                k = g * GRP + j
                pltpu.make_async_copy(vals_buf.at[pl.ds(k * CHUNK, CHUNK)],
                                      target.at[row_buf.at[k]], sem_scat).wait()
                return c2
            lax.fori_loop(0, GRP, drain_body, 0)
            return c
        lax.fori_loop(0, NGRP, grp_body, 0)

    _scatter_all(sp_deg)

    plsc.subcore_barrier()

    pltpu.sync_copy(sp_deg.at[pl.ds(nbase, NPT)], slice_buf)

    def dbody(i, c):
        d = slice_buf[pl.ds(i * 16, 16)]
        dinv_slice[pl.ds(i * 16, 16)] = _rsqrt_newton(d)
        return c
    lax.fori_loop(0, NPT // 16, dbody, 0)
    pltpu.sync_copy(dinv_slice, sp_dinv.at[pl.ds(nbase, NPT)])

    plsc.subcore_barrier()

    pltpu.sync_copy(sp_dinv, dinv_full)

    def gbody(i, c):
        cidx = col_buf[pl.ds(i * 16, 16)]
        vals_buf[pl.ds(i * 16, 16)] = plsc.load_gather(dinv_full, [cidx])
        return c
    lax.fori_loop(0, EPTP // 16, gbody, 0)

    _scatter_all(sp_t)

    plsc.subcore_barrier()

    pltpu.sync_copy(sp_t.at[pl.ds(nbase, NPT)], slice_buf)

    def fbody(i, c):
        t = slice_buf[pl.ds(i * 16, 16)]
        dv = dinv_slice[pl.ds(i * 16, 16)]
        slice_buf[pl.ds(i * 16, 16)] = t * dv
        return c
    lax.fori_loop(0, NPT // 16, fbody, 0)
    pltpu.sync_copy(slice_buf, out_hbm.at[cid, pl.ds(nbase, NPT)])


def _tc_body(x_ref, win_ref, wout_ref, wloop_ref, sin_ref, sout_ref,
             g_ref, b_ref, o_ref):
    x = x_ref[...]
    pre = (jnp.dot(x, win_ref[...], preferred_element_type=jnp.float32) * sin_ref[...]
           + jnp.dot(x, wout_ref[...], preferred_element_type=jnp.float32) * sout_ref[...]
           + jnp.dot(x, wloop_ref[...], preferred_element_type=jnp.float32)
           ) * jnp.float32(1.0 / 3.0)
    mean = jnp.mean(pre, axis=0, keepdims=True)
    var = jnp.mean(pre * pre, axis=0, keepdims=True) - mean * mean
    inv = lax.rsqrt(var + 1e-5)
    o_ref[...] = jnp.tanh(g_ref[...] * (pre - mean) * inv + b_ref[...])


def kernel(batch, x, edge_index, rel_embed, W_in, W_out, W_loop, gamma, beta):
    ei4 = edge_index.reshape(2, NCORE, NSUB, EPT)
    eip = jnp.pad(ei4, ((0, 0), (0, 0), (0, 0), (0, EPTP - EPT)),
                  constant_values=PADIDX)
    row_t = eip[0].reshape(NCORE, NSUB, NCHUNK, CHUNK)
    col_t = eip[1]

    s2 = _sc_coeffs(row_t, col_t)
    sin = s2[0, :N].reshape(N, 1)
    sout = s2[1, :N].reshape(N, 1)

    out = pl.pallas_call(
        _tc_body,
        out_shape=jax.ShapeDtypeStruct((N, D), jnp.float32),
    )(x, W_in, W_out, W_loop, sin, sout, gamma.reshape(1, D), beta.reshape(1, D))
    return out, rel_embed
